# baseline (speedup 1.0000x reference)
import jax
import jax.numpy as jnp
from jax.experimental import pallas as pl
from jax.experimental.pallas import tpu as pltpu

TOPK_E = 16
TOPK_D = 2048
HALF_BLK = 512


def _router_body(x1_ref, x2_ref, w_ref, idx1_ref, idx2_ref, p1_ref, p2_ref,
                 cv_ref, acc_ref):
    step = pl.program_id(0)
    nsteps = pl.num_programs(0)

    w = w_ref[...]
    l1 = jax.lax.dot_general(
        w, x1_ref[...], (((1,), (1,)), ((), ())),
        preferred_element_type=jnp.float32)
    l2 = jax.lax.dot_general(
        w, x2_ref[...], (((1,), (1,)), ((), ())),
        preferred_element_type=jnp.float32)
    logits = jnp.concatenate([l1, l2], axis=1)

    blk = logits.shape[1]
    e_iota = jax.lax.broadcasted_iota(jnp.int32, (TOPK_E, blk), 0)

    m1 = jnp.max(logits, axis=0, keepdims=True)
    i1 = jnp.min(jnp.where(logits == m1, e_iota, TOPK_E),
                 axis=0, keepdims=True)
    masked = jnp.where(e_iota == i1, -jnp.inf, logits)
    m2 = jnp.max(masked, axis=0, keepdims=True)
    i2 = jnp.min(jnp.where(masked == m2, e_iota, TOPK_E),
                 axis=0, keepdims=True)

    t = jnp.exp(m2 - m1)
    denom = 1.0 + t
    pa = 1.0 / denom
    pb = t / denom

    idx = jnp.concatenate([i1, i2], axis=0)
    probs = jnp.concatenate([pa, pb], axis=0)
    idx1_ref[...] = idx[:, :HALF_BLK]
    idx2_ref[...] = idx[:, HALF_BLK:]
    p1_ref[...] = probs[:, :HALF_BLK]
    p2_ref[...] = probs[:, HALF_BLK:]

    ex = jnp.exp(logits - m1)
    gp = ex / jnp.sum(ex, axis=0, keepdims=True)
    part = gp.reshape(TOPK_E, blk // 128, 128).sum(axis=1)

    @pl.when(step == 0)
    def _init():
        acc_ref[...] = part

    @pl.when(step != 0)
    def _acc():
        acc_ref[...] += part

    @pl.when(step == nsteps - 1)
    def _finalize():
        n_tokens = jnp.float32(nsteps * blk)
        mean_probs = jnp.sum(acc_ref[...], axis=1, keepdims=True) / n_tokens
        mu = jnp.mean(mean_probs)
        var = jnp.sum((mean_probs - mu) ** 2) / jnp.float32(TOPK_E - 1)
        cv = var / (mu + 1e-10) ** 2
        cv_ref[...] = jnp.broadcast_to(cv, (1, 1))


def kernel(x, W):
    b, s, d = x.shape
    n = b * s
    h = n // 2
    x_flat = x.reshape(n, d)
    x1, x2 = x_flat[:h], x_flat[h:]
    grid = h // HALF_BLK

    idx_a, idx_b, probs_a, probs_b, cv = pl.pallas_call(
        _router_body,
        grid=(grid,),
        in_specs=[
            pl.BlockSpec((HALF_BLK, d), lambda i: (i, 0)),
            pl.BlockSpec((HALF_BLK, d), lambda i: (i, 0)),
            pl.BlockSpec((TOPK_E, d), lambda i: (0, 0)),
        ],
        out_specs=[
            pl.BlockSpec((2, HALF_BLK), lambda i: (0, i)),
            pl.BlockSpec((2, HALF_BLK), lambda i: (0, i)),
            pl.BlockSpec((2, HALF_BLK), lambda i: (0, i)),
            pl.BlockSpec((2, HALF_BLK), lambda i: (0, i)),
            pl.BlockSpec((1, 1), lambda i: (0, 0)),
        ],
        out_shape=[
            jax.ShapeDtypeStruct((2, h), jnp.int32),
            jax.ShapeDtypeStruct((2, h), jnp.int32),
            jax.ShapeDtypeStruct((2, h), jnp.float32),
            jax.ShapeDtypeStruct((2, h), jnp.float32),
            jax.ShapeDtypeStruct((1, 1), jnp.float32),
        ],
        scratch_shapes=[pltpu.VMEM((TOPK_E, 128), jnp.float32)],
        compiler_params=pltpu.CompilerParams(
            dimension_semantics=("arbitrary",),
        ),
    )(x1, x2, W)

    idx = jnp.concatenate([idx_a, idx_b], axis=1).T
    probs = jnp.concatenate([probs_a, probs_b], axis=1).T
    return (idx, probs, cv.reshape(()))


# speedup vs baseline: 1.5736x; 1.5736x over previous
import jax
import jax.numpy as jnp
from jax import lax
from jax.experimental import pallas as pl
from jax.experimental.pallas import tpu as pltpu
from jax.experimental.pallas import tpu_sc as plsc

TOPK_E = 16
TOKEN_BLK = 1024
NC, NS, L = 2, 16, 16
NW = NC * NS


def _matmul_cv_body(x_ref, w_ref, lg_ref, cv_ref, acc_ref):
    step = pl.program_id(0)
    nsteps = pl.num_programs(0)

    logits = jax.lax.dot_general(
        w_ref[...], x_ref[...], (((1,), (1,)), ((), ())),
        preferred_element_type=jnp.float32)
    lg_ref[...] = logits

    blk = logits.shape[1]
    m1 = jnp.max(logits, axis=0, keepdims=True)
    ex = jnp.exp(logits - m1)
    gp = ex / jnp.sum(ex, axis=0, keepdims=True)
    part = gp.reshape(TOPK_E, blk // 128, 128).sum(axis=1)

    @pl.when(step == 0)
    def _init():
        acc_ref[...] = part

    @pl.when(step != 0)
    def _acc():
        acc_ref[...] += part

    @pl.when(step == nsteps - 1)
    def _finalize():
        n_tokens = jnp.float32(nsteps * blk)
        mean_probs = jnp.sum(acc_ref[...], axis=1, keepdims=True) / n_tokens
        mu = jnp.mean(mean_probs)
        var = jnp.sum((mean_probs - mu) ** 2) / jnp.float32(TOPK_E - 1)
        cv = var / (mu + 1e-10) ** 2
        cv_ref[...] = jnp.broadcast_to(cv, (1, 1))


def _sc_top2_body(lg_hbm, i1_hbm, i2_hbm, p1_hbm, p2_hbm,
                  lg_v, i1_v, i2_v, p1_v, p2_v):
    c = lax.axis_index("c")
    s = lax.axis_index("s")
    wid = s * NC + c
    n = lg_hbm.shape[1]
    tpt = n // NW
    g_cnt = tpt // L
    base = wid * tpt

    pltpu.sync_copy(lg_hbm.at[:, pl.ds(base, tpt)], lg_v)
    minf = jnp.float32(-jnp.inf)

    for g in range(g_cnt):
        lv = [lg_v[e, pl.ds(g * L, L)] for e in range(TOPK_E)]
        m1 = lv[0]
        for e in range(1, TOPK_E):
            m1 = jnp.maximum(m1, lv[e])
        i1 = jnp.full((L,), TOPK_E - 1, jnp.int32)
        for e in range(TOPK_E - 2, -1, -1):
            i1 = jnp.where(lv[e] == m1, jnp.int32(e), i1)
        l2 = [jnp.where(i1 == e, minf, lv[e]) for e in range(TOPK_E)]
        m2 = l2[0]
        for e in range(1, TOPK_E):
            m2 = jnp.maximum(m2, l2[e])
        i2 = jnp.full((L,), TOPK_E - 1, jnp.int32)
        for e in range(TOPK_E - 2, -1, -1):
            i2 = jnp.where(l2[e] == m2, jnp.int32(e), i2)

        t = jnp.exp(m2 - m1)
        den = 1.0 + t
        i1_v[pl.ds(g * L, L)] = i1
        i2_v[pl.ds(g * L, L)] = i2
        p1_v[pl.ds(g * L, L)] = 1.0 / den
        p2_v[pl.ds(g * L, L)] = t / den

    pltpu.sync_copy(i1_v, i1_hbm.at[pl.ds(base, tpt)])
    pltpu.sync_copy(i2_v, i2_hbm.at[pl.ds(base, tpt)])
    pltpu.sync_copy(p1_v, p1_hbm.at[pl.ds(base, tpt)])
    pltpu.sync_copy(p2_v, p2_hbm.at[pl.ds(base, tpt)])


def kernel(x, W):
    b, s, d = x.shape
    n = b * s
    x_flat = x.reshape(n, d)
    grid = n // TOKEN_BLK

    logits_t, cv = pl.pallas_call(
        _matmul_cv_body,
        grid=(grid,),
        in_specs=[
            pl.BlockSpec((TOKEN_BLK, d), lambda i: (i, 0)),
            pl.BlockSpec((TOPK_E, d), lambda i: (0, 0)),
        ],
        out_specs=[
            pl.BlockSpec((TOPK_E, TOKEN_BLK), lambda i: (0, i)),
            pl.BlockSpec((1, 1), lambda i: (0, 0)),
        ],
        out_shape=[
            jax.ShapeDtypeStruct((TOPK_E, n), jnp.float32),
            jax.ShapeDtypeStruct((1, 1), jnp.float32),
        ],
        scratch_shapes=[pltpu.VMEM((TOPK_E, 128), jnp.float32)],
        compiler_params=pltpu.CompilerParams(
            dimension_semantics=("arbitrary",),
        ),
    )(x_flat, W)

    tpt = n // NW
    mesh = plsc.VectorSubcoreMesh(core_axis_name="c", subcore_axis_name="s")
    i1_flat, i2_flat, p1_flat, p2_flat = pl.kernel(
        _sc_top2_body,
        out_type=[
            jax.ShapeDtypeStruct((n,), jnp.int32),
            jax.ShapeDtypeStruct((n,), jnp.int32),
            jax.ShapeDtypeStruct((n,), jnp.float32),
            jax.ShapeDtypeStruct((n,), jnp.float32),
        ],
        mesh=mesh,
        scratch_types=[
            pltpu.VMEM((TOPK_E, tpt), jnp.float32),
            pltpu.VMEM((tpt,), jnp.int32),
            pltpu.VMEM((tpt,), jnp.int32),
            pltpu.VMEM((tpt,), jnp.float32),
            pltpu.VMEM((tpt,), jnp.float32),
        ],
    )(logits_t)

    idx = jnp.stack([i1_flat, i2_flat], axis=1)
    probs = jnp.stack([p1_flat, p2_flat], axis=1)
    return (idx, probs, cv.reshape(()))


# speedup vs baseline: 3.0396x; 1.9316x over previous
import jax
import jax.numpy as jnp
from jax.experimental import pallas as pl
from jax.experimental.pallas import tpu as pltpu

TOPK_E = 16
TOPK_D = 2048
TOKEN_BLK = 1024


def _router_body(x1_ref, x2_ref, w_ref, idx_ref, probs_ref, cv_ref, acc_ref):
    step = pl.program_id(0)
    nsteps = pl.num_programs(0)

    w = w_ref[...]
    l1 = jax.lax.dot_general(
        w, x1_ref[...], (((1,), (1,)), ((), ())),
        preferred_element_type=jnp.float32)
    l2 = jax.lax.dot_general(
        w, x2_ref[...], (((1,), (1,)), ((), ())),
        preferred_element_type=jnp.float32)
    logits = jnp.concatenate([l1, l2], axis=1)

    blk = logits.shape[1]
    e_iota = jax.lax.broadcasted_iota(jnp.int32, (TOPK_E, blk), 0)

    m1 = jnp.max(logits, axis=0, keepdims=True)
    i1 = jnp.min(jnp.where(logits == m1, e_iota, TOPK_E),
                 axis=0, keepdims=True)
    masked = jnp.where(e_iota == i1, -jnp.inf, logits)
    m2 = jnp.max(masked, axis=0, keepdims=True)
    i2 = jnp.min(jnp.where(masked == m2, e_iota, TOPK_E),
                 axis=0, keepdims=True)

    t = jnp.exp(m2 - m1)
    denom = 1.0 + t
    p1 = 1.0 / denom
    p2 = t / denom

    idx_ref[...] = jnp.concatenate([i1, i2], axis=0)
    probs_ref[...] = jnp.concatenate([p1, p2], axis=0)

    ex = jnp.exp(logits - m1)
    gp = ex / jnp.sum(ex, axis=0, keepdims=True)
    part = gp.reshape(TOPK_E, blk // 128, 128).sum(axis=1)

    @pl.when(step == 0)
    def _init():
        acc_ref[...] = part

    @pl.when(step != 0)
    def _acc():
        acc_ref[...] += part

    @pl.when(step == nsteps - 1)
    def _finalize():
        n_tokens = jnp.float32(nsteps * blk)
        mean_probs = jnp.sum(acc_ref[...], axis=1, keepdims=True) / n_tokens
        mu = jnp.mean(mean_probs)
        var = jnp.sum((mean_probs - mu) ** 2) / jnp.float32(TOPK_E - 1)
        cv = var / (mu + 1e-10) ** 2
        cv_ref[...] = jnp.broadcast_to(cv, (1, 1))


def kernel(x, W):
    b, s, d = x.shape
    n = b * s
    x_flat = x.reshape(n, d)
    grid = n // TOKEN_BLK

    idx_t, probs_t, cv = pl.pallas_call(
        _router_body,
        grid=(grid,),
        in_specs=[
            pl.BlockSpec((TOKEN_BLK // 2, d), lambda i: (2 * i, 0)),
            pl.BlockSpec((TOKEN_BLK // 2, d), lambda i: (2 * i + 1, 0)),
            pl.BlockSpec((TOPK_E, d), lambda i: (0, 0)),
        ],
        out_specs=[
            pl.BlockSpec((2, TOKEN_BLK), lambda i: (0, i)),
            pl.BlockSpec((2, TOKEN_BLK), lambda i: (0, i)),
            pl.BlockSpec((1, 1), lambda i: (0, 0)),
        ],
        out_shape=[
            jax.ShapeDtypeStruct((2, n), jnp.int32),
            jax.ShapeDtypeStruct((2, n), jnp.float32),
            jax.ShapeDtypeStruct((1, 1), jnp.float32),
        ],
        scratch_shapes=[pltpu.VMEM((TOPK_E, 128), jnp.float32)],
        compiler_params=pltpu.CompilerParams(
            dimension_semantics=("arbitrary",),
        ),
    )(x_flat, x_flat, W)

    return (idx_t.T, probs_t.T, cv.reshape(()))


# speedup vs baseline: 3.0628x; 1.0076x over previous
import jax
import jax.numpy as jnp
from jax.experimental import pallas as pl
from jax.experimental.pallas import tpu as pltpu

TOPK_E = 16
TOPK_D = 2048
TOKEN_BLK = 1024


def _router_body(x_ref, w_ref, idx_ref, probs_ref, cv_ref, acc_ref):
    step = pl.program_id(0)
    nsteps = pl.num_programs(0)

    x_blk = x_ref[...]
    w = w_ref[...]
    logits = jax.lax.dot_general(
        w, x_blk, (((1,), (1,)), ((), ())),
        preferred_element_type=jnp.float32)

    blk = logits.shape[1]
    e_iota = jax.lax.broadcasted_iota(jnp.int32, (TOPK_E, blk), 0)

    m1 = jnp.max(logits, axis=0, keepdims=True)
    i1 = jnp.min(jnp.where(logits == m1, e_iota, TOPK_E),
                 axis=0, keepdims=True)
    masked = jnp.where(e_iota == i1, -jnp.inf, logits)
    m2 = jnp.max(masked, axis=0, keepdims=True)
    i2 = jnp.min(jnp.where(masked == m2, e_iota, TOPK_E),
                 axis=0, keepdims=True)

    t = jnp.exp(m2 - m1)
    denom = 1.0 + t
    p1 = 1.0 / denom
    p2 = t / denom

    idx_ref[...] = jnp.concatenate([i1, i2], axis=0)
    probs_ref[...] = jnp.concatenate([p1, p2], axis=0)

    ex = jnp.exp(logits - m1)
    gp = ex / jnp.sum(ex, axis=0, keepdims=True)
    part = gp.reshape(TOPK_E, blk // 128, 128).sum(axis=1)

    @pl.when(step == 0)
    def _init():
        acc_ref[...] = part

    @pl.when(step != 0)
    def _acc():
        acc_ref[...] += part

    @pl.when(step == nsteps - 1)
    def _finalize():
        n_tokens = jnp.float32(nsteps * blk)
        mean_probs = jnp.sum(acc_ref[...], axis=1, keepdims=True) / n_tokens
        mu = jnp.mean(mean_probs)
        var = jnp.sum((mean_probs - mu) ** 2) / jnp.float32(TOPK_E - 1)
        cv = var / (mu + 1e-10) ** 2
        cv_ref[...] = jnp.broadcast_to(cv, (1, 1))


def kernel(x, W):
    b, s, d = x.shape
    n = b * s
    x_flat = x.reshape(n, d)
    grid = n // TOKEN_BLK

    idx_t, probs_t, cv = pl.pallas_call(
        _router_body,
        grid=(grid,),
        in_specs=[
            pl.BlockSpec((TOKEN_BLK, d), lambda i: (i, 0)),
            pl.BlockSpec((TOPK_E, d), lambda i: (0, 0)),
        ],
        out_specs=[
            pl.BlockSpec((2, TOKEN_BLK), lambda i: (0, i)),
            pl.BlockSpec((2, TOKEN_BLK), lambda i: (0, i)),
            pl.BlockSpec((1, 1), lambda i: (0, 0)),
        ],
        out_shape=[
            jax.ShapeDtypeStruct((2, n), jnp.int32),
            jax.ShapeDtypeStruct((2, n), jnp.float32),
            jax.ShapeDtypeStruct((1, 1), jnp.float32),
        ],
        scratch_shapes=[pltpu.VMEM((TOPK_E, 128), jnp.float32)],
        compiler_params=pltpu.CompilerParams(
            dimension_semantics=("arbitrary",),
        ),
    )(x_flat, W)

    return (idx_t.T, probs_t.T, cv.reshape(()))
